# baseline (speedup 1.0000x reference)
import functools
import jax
import jax.numpy as jnp
from jax import lax
from jax.experimental import pallas as pl
from jax.experimental.pallas import tpu as pltpu
from jax.experimental.pallas import tpu_sc as plsc


def _lane_perm(v, perm_idx):
    return lax.gather(
        v,
        perm_idx[:, None],
        lax.GatherDimensionNumbers(
            offset_dims=(), collapsed_slice_dims=(0,), start_index_map=(0,)
        ),
        slice_sizes=(1,),
        mode=lax.GatherScatterMode.PROMISE_IN_BOUNDS,
    )


def kernel(batch_mask, mask_emb):
    M, N = batch_mask.shape
    _, D = mask_emb.shape
    B = M * N
    NC, NS, L = 2, 16, 16
    NW = NC * NS
    P = B // 2
    p_per_w = P // NW
    K = 2
    PCH = 128 * K
    ICH = 2 * PCH
    n_ch = p_per_w // PCH

    idx = batch_mask.reshape(B)

    combo = jnp.concatenate(
        [
            jnp.concatenate([mask_emb[c >> 1], mask_emb[c & 1]])[None, :]
            for c in range(4)
        ],
        axis=0,
    )

    mesh = plsc.VectorSubcoreMesh(
        core_axis_name="c", subcore_axis_name="s", num_cores=NC, num_subcores=NS
    )

    @functools.partial(
        pl.kernel,
        mesh=mesh,
        out_type=jax.ShapeDtypeStruct((P, 2 * D), jnp.float32),
        scratch_types=[
            pltpu.VMEM((ICH,), jnp.int32),
            pltpu.VMEM((ICH,), jnp.int32),
            pltpu.VMEM((K, 128), jnp.int32),
            pltpu.VMEM((K, 128), jnp.int32),
            pltpu.VMEM((PCH, 2 * D), jnp.float32),
            pltpu.VMEM((PCH, 2 * D), jnp.float32),
            pltpu.SemaphoreType.DMA,
            pltpu.SemaphoreType.DMA,
            pltpu.SemaphoreType.DMA,
            pltpu.SemaphoreType.DMA,
        ],
    )
    def k(combo_hbm, idx_hbm, out_hbm, i_v0, i_v1, c_v0, c_v1, r_v0, r_v1,
          sg0, sg1, so0, so1):
        wid = lax.axis_index("s") * NC + lax.axis_index("c")
        ibase = wid * 2 * p_per_w
        obase = wid * p_per_w

        ii = lax.iota(jnp.int32, L)
        rot1 = (ii + 1) % L
        even2 = (2 * ii) % L
        lo8 = ii < 8

        bufs = ((i_v0, c_v0, r_v0, sg0, so0), (i_v1, c_v1, r_v1, sg1, so1))

        def half(i, b, first):
            i_v, c_v, r_v, sg, so = bufs[b]
            if not first:
                pltpu.make_async_copy(
                    r_v, out_hbm.at[pl.ds(obase, PCH)], so
                ).wait()
            pltpu.sync_copy(idx_hbm.at[pl.ds(ibase + i * ICH, ICH)], i_v)
            for q in range(PCH // L):
                w0 = i_v[pl.ds(2 * L * q, L)]
                w1 = i_v[pl.ds(2 * L * q + L, L)]
                cc0 = 2 * w0 + _lane_perm(w0, rot1)
                cc1 = 2 * w1 + _lane_perm(w1, rot1)
                z = jnp.where(lo8, _lane_perm(cc0, even2), _lane_perm(cc1, even2))
                c_v[(L * q) // 128, pl.ds((L * q) % 128, L)] = z
            for r in range(K):
                pltpu.async_copy(
                    combo_hbm.at[c_v.at[r]],
                    r_v.at[pl.ds(128 * r, 128)],
                    sg,
                )
            pltpu.make_async_copy(combo_hbm.at[c_v.at[0]], r_v, sg).wait()
            pltpu.async_copy(r_v, out_hbm.at[pl.ds(obase + i * PCH, PCH)], so)

        half(0, 0, True)
        half(1, 1, True)

        def step(j, carry):
            half(2 * j, 0, False)
            half(2 * j + 1, 1, False)
            return carry

        lax.fori_loop(1, n_ch // 2, step, 0)

        for b in range(2):
            i_v, c_v, r_v, sg, so = bufs[b]
            pltpu.make_async_copy(
                r_v, out_hbm.at[pl.ds(obase, PCH)], so
            ).wait()

    out = k(combo, idx)
    return out.reshape(M, N, D)


# speedup vs baseline: 7.7323x; 7.7323x over previous
import functools
import jax
import jax.numpy as jnp
from jax import lax
from jax.experimental import pallas as pl
from jax.experimental.pallas import tpu as pltpu
from jax.experimental.pallas import tpu_sc as plsc


def _lane_perm(v, perm_idx):
    return lax.gather(
        v,
        perm_idx[:, None],
        lax.GatherDimensionNumbers(
            offset_dims=(), collapsed_slice_dims=(0,), start_index_map=(0,)
        ),
        slice_sizes=(1,),
        mode=lax.GatherScatterMode.PROMISE_IN_BOUNDS,
    )


def kernel(batch_mask, mask_emb):
    M, N = batch_mask.shape
    _, D = mask_emb.shape
    B = M * N
    NC, NS, L = 2, 16, 16
    NW = NC * NS
    P = B // 2
    p_per_w = P // NW
    K = 2
    PCH = 128 * K
    ICH = 2 * PCH
    n_ch = p_per_w // PCH

    idx = batch_mask.reshape(B)

    combo = jnp.concatenate(
        [
            jnp.concatenate([mask_emb[c >> 1], mask_emb[c & 1]])[None, :]
            for c in range(4)
        ],
        axis=0,
    )

    mesh = plsc.VectorSubcoreMesh(
        core_axis_name="c", subcore_axis_name="s", num_cores=NC, num_subcores=NS
    )

    @functools.partial(
        pl.kernel,
        mesh=mesh,
        out_type=jax.ShapeDtypeStruct((P, 2 * D), jnp.float32),
        scratch_types=[
            pltpu.VMEM((ICH,), jnp.int32),
            pltpu.VMEM((ICH,), jnp.int32),
            pltpu.VMEM((K, 128), jnp.int32),
            pltpu.VMEM((K, 128), jnp.int32),
            pltpu.VMEM((PCH, 2 * D), jnp.float32),
            pltpu.VMEM((PCH, 2 * D), jnp.float32),
            pltpu.VMEM_SHARED((4, 2 * D), jnp.float32),
            pltpu.SemaphoreType.DMA,
            pltpu.SemaphoreType.DMA,
            pltpu.SemaphoreType.DMA,
            pltpu.SemaphoreType.DMA,
        ],
    )
    def k(combo_hbm, idx_hbm, out_hbm, i_v0, i_v1, c_v0, c_v1, r_v0, r_v1,
          combo_v, sg0, sg1, so0, so1):
        wid = lax.axis_index("s") * NC + lax.axis_index("c")

        @pl.when(lax.axis_index("s") == 0)
        def _stage_table():
            pltpu.sync_copy(combo_hbm, combo_v)

        plsc.subcore_barrier()
        ibase = wid * 2 * p_per_w
        obase = wid * p_per_w

        ii = lax.iota(jnp.int32, L)
        rot1 = (ii + 1) % L
        even2 = (2 * ii) % L
        lo8 = ii < 8

        bufs = ((i_v0, c_v0, r_v0, sg0, so0), (i_v1, c_v1, r_v1, sg1, so1))

        def half(i, b, first):
            i_v, c_v, r_v, sg, so = bufs[b]
            if not first:
                pltpu.make_async_copy(
                    r_v, out_hbm.at[pl.ds(obase, PCH)], so
                ).wait()
            pltpu.sync_copy(idx_hbm.at[pl.ds(ibase + i * ICH, ICH)], i_v)
            for q in range(PCH // L):
                w0 = i_v[pl.ds(2 * L * q, L)]
                w1 = i_v[pl.ds(2 * L * q + L, L)]
                cc0 = 2 * w0 + _lane_perm(w0, rot1)
                cc1 = 2 * w1 + _lane_perm(w1, rot1)
                z = jnp.where(lo8, _lane_perm(cc0, even2), _lane_perm(cc1, even2))
                c_v[(L * q) // 128, pl.ds((L * q) % 128, L)] = z
            for r in range(K):
                pltpu.async_copy(
                    combo_v.at[c_v.at[r]],
                    r_v.at[pl.ds(128 * r, 128)],
                    sg,
                )
            pltpu.make_async_copy(combo_v.at[c_v.at[0]], r_v, sg).wait()
            pltpu.async_copy(r_v, out_hbm.at[pl.ds(obase + i * PCH, PCH)], so)

        half(0, 0, True)
        half(1, 1, True)

        def step(j, carry):
            half(2 * j, 0, False)
            half(2 * j + 1, 1, False)
            return carry

        lax.fori_loop(1, n_ch // 2, step, 0)

        for b in range(2):
            i_v, c_v, r_v, sg, so = bufs[b]
            pltpu.make_async_copy(
                r_v, out_hbm.at[pl.ds(obase, PCH)], so
            ).wait()

    out = k(combo, idx)
    return out.reshape(M, N, D)
